# baseline (speedup 1.0000x reference)
import functools

import jax
import jax.numpy as jnp
from jax import lax
from jax.experimental import pallas as pl
from jax.experimental.pallas import tpu as pltpu
from jax.experimental.pallas import tpu_sc as plsc

HIDDEN = 1024
NC = 2
NS = 16
NW = NC * NS
CHUNK = 56


def _make_gather(ntok: int):
    assert ntok % NW == 0
    bpw = ntok // NW
    sizes = [CHUNK] * (bpw // CHUNK)
    if bpw % CHUNK:
        sizes.append(bpw % CHUNK)
    offs = [sum(sizes[:i]) for i in range(len(sizes))]
    nch = len(sizes)
    assert nch >= 2

    mesh = plsc.VectorSubcoreMesh(core_axis_name="c", subcore_axis_name="s")

    @functools.partial(
        pl.kernel,
        mesh=mesh,
        out_type=jax.ShapeDtypeStruct((ntok, HIDDEN), jnp.float32),
        scratch_types=[
            pltpu.VMEM((bpw,), jnp.int32),
            [pltpu.VMEM((CHUNK, HIDDEN), jnp.float32) for _ in range(2)],
            [pltpu.SemaphoreType.DMA for _ in range(2)],
            [pltpu.SemaphoreType.DMA for _ in range(2)],
        ],
    )
    def gather_kernel(ids_hbm, table_hbm, out_hbm, idx_v, bufs, gsems, ssems):
        wid = lax.axis_index("s") * NC + lax.axis_index("c")
        base = wid * bpw
        pltpu.sync_copy(ids_hbm.at[pl.ds(base, bpw)], idx_v)

        def start_gather(ch, b):
            n = sizes[ch]
            pltpu.async_copy(
                table_hbm.at[idx_v.at[pl.ds(offs[ch], n)]],
                bufs[b].at[pl.ds(0, n)],
                gsems[b],
            )

        def wait_gather(ch, b):
            n = sizes[ch]
            pltpu.make_async_copy(
                table_hbm.at[idx_v.at[pl.ds(0, n)]], bufs[b].at[pl.ds(0, n)], gsems[b]
            ).wait()

        def start_store(ch, b):
            n = sizes[ch]
            pltpu.async_copy(
                bufs[b].at[pl.ds(0, n)],
                out_hbm.at[pl.ds(base + offs[ch], n)],
                ssems[b],
            )

        def wait_store(ch, b):
            n = sizes[ch]
            pltpu.make_async_copy(
                bufs[b].at[pl.ds(0, n)], out_hbm.at[pl.ds(base, n)], ssems[b]
            ).wait()

        start_gather(0, 0)
        start_gather(1, 1)
        for ch in range(nch):
            b = ch % 2
            wait_gather(ch, b)
            start_store(ch, b)
            wait_store(ch, b)
            if ch + 2 < nch:
                start_gather(ch + 2, b)

    return gather_kernel


def kernel(input_ids, position_ids, embed_tokens):
    batch, seq = input_ids.shape
    ids_flat = input_ids.reshape(-1)
    rows = _make_gather(batch * seq)(ids_flat, embed_tokens)
    hidden_states = rows.reshape(batch, seq, HIDDEN)
    return hidden_states, position_ids
